# baseline (speedup 1.0000x reference)
import jax
import jax.numpy as jnp
from jax import lax
from jax.experimental import pallas as pl
from jax.experimental.pallas import tpu as pltpu
from jax.experimental.pallas import tpu_sc as plsc

_B = 4096
_DIM = 64
_NW = 32
_BPW = _B // _NW
_L = 16
_TN = 256


def _sc_gather(user_h, item_h, ut_h, it_h, ib2_h, gu_h, gi_h, beta_h,
               idxu_v, idxi_v, brow_idx_v, ru_v, ri_v, brows_v, beta_v,
               s1, s2, s3):
    wid = lax.axis_index("s") * 2 + lax.axis_index("c")
    base = wid * _BPW
    pltpu.sync_copy(user_h.at[pl.ds(base, _BPW)], idxu_v)
    pltpu.sync_copy(item_h.at[pl.ds(base, _BPW)], idxi_v)
    for g in range(_BPW // _L):
        v = idxi_v[pl.ds(g * _L, _L)]
        brow_idx_v[pl.ds(g * _L, _L)] = lax.shift_right_logical(v, 4)
    cu = pltpu.async_copy(ut_h.at[idxu_v], ru_v, s1)
    ci = pltpu.async_copy(it_h.at[idxi_v], ri_v, s2)
    cb = pltpu.async_copy(ib2_h.at[brow_idx_v], brows_v, s3)
    cu.wait()
    ci.wait()
    cb.wait()
    for g in range(_BPW // _L):
        rid = lax.iota(jnp.int32, _L) + g * _L
        lane = lax.bitwise_and(idxi_v[pl.ds(g * _L, _L)], 15)
        beta_v[pl.ds(g * _L, _L)] = plsc.load_gather(brows_v, [rid, lane])
    pltpu.sync_copy(ru_v, gu_h.at[pl.ds(base, _BPW)])
    pltpu.sync_copy(ri_v, gi_h.at[pl.ds(base, _BPW)])
    pltpu.sync_copy(beta_v, beta_h.at[pl.ds(base, _BPW)])


def _tc_xui(beta_ref, gu_ref, gi_ref, out_ref):
    s = jnp.sum(gu_ref[...] * gi_ref[...], axis=1)
    out_ref[...] = beta_ref[...] + s[None, :]


def kernel(user, item, user_table, item_table, item_bias):
    n_items = item_bias.shape[0]
    ib2 = item_bias.reshape(n_items // _L, _L)
    mesh = plsc.VectorSubcoreMesh(core_axis_name="c", subcore_axis_name="s")
    gather = pl.kernel(
        _sc_gather,
        mesh=mesh,
        out_type=[
            jax.ShapeDtypeStruct((_B, _DIM), jnp.float32),
            jax.ShapeDtypeStruct((_B, _DIM), jnp.float32),
            jax.ShapeDtypeStruct((_B,), jnp.float32),
        ],
        scratch_types=[
            pltpu.VMEM((_BPW,), jnp.int32),
            pltpu.VMEM((_BPW,), jnp.int32),
            pltpu.VMEM((_BPW,), jnp.int32),
            pltpu.VMEM((_BPW, _DIM), jnp.float32),
            pltpu.VMEM((_BPW, _DIM), jnp.float32),
            pltpu.VMEM((_BPW, _L), jnp.float32),
            pltpu.VMEM((_BPW,), jnp.float32),
            pltpu.SemaphoreType.DMA,
            pltpu.SemaphoreType.DMA,
            pltpu.SemaphoreType.DMA,
        ],
        compiler_params=pltpu.CompilerParams(use_tc_tiling_on_sc=False,
                                             needs_layout_passes=False),
    )
    gamma_u, gamma_i, beta_flat = gather(user, item, user_table, item_table,
                                         ib2)
    beta_i = beta_flat.reshape(_B, 1)

    xui = pl.pallas_call(
        _tc_xui,
        grid=(_B // _TN,),
        in_specs=[
            pl.BlockSpec((_B, 1), lambda j: (0, 0)),
            pl.BlockSpec((_TN, _DIM), lambda j: (j, 0)),
            pl.BlockSpec((_TN, _DIM), lambda j: (j, 0)),
        ],
        out_specs=pl.BlockSpec((_B, _TN), lambda j: (0, j)),
        out_shape=jax.ShapeDtypeStruct((_B, _B), jnp.float32),
    )(beta_i, gamma_u, gamma_i)

    return (xui, beta_i, gamma_u, gamma_i)
